# baseline (speedup 1.0000x reference)
import functools

import jax
import jax.numpy as jnp
from jax import lax
from jax.experimental import pallas as pl
from jax.experimental.pallas import tpu as pltpu
from jax.experimental.pallas import tpu_sc as plsc

TOP_RATIO = 0.3
TOP_WEIGHT = 1.0
IGNORE_INDEX = -1

_PIX_BLK = 8192


def _nll_body(pred_ref, gt_ref, nll_ref, stats_ref, acc_ref):
    i = pl.program_id(0)
    n = pl.num_programs(0)
    x = pred_ref[0]
    g = gt_ref[0]
    s = jnp.sum(jnp.exp(x), axis=0)
    lse = jnp.log(s)
    cio = lax.broadcasted_iota(jnp.int32, x.shape, 0)
    xg = jnp.sum(jnp.where(cio == g[None], x, 0.0), axis=0)
    valid = g != IGNORE_INDEX
    nll = jnp.where(valid, jnp.maximum(lse - xg, 0.0), 0.0)
    nll_ref[0] = nll

    @pl.when(i == 0)
    def _():
        acc_ref[0] = jnp.zeros_like(acc_ref[0])
        acc_ref[1] = jnp.zeros_like(acc_ref[1])

    acc_ref[0] += nll
    acc_ref[1] += valid.astype(jnp.float32)

    @pl.when(i == n - 1)
    def _():
        stats_ref[0, 0] = jnp.sum(acc_ref[0])
        stats_ref[0, 1] = jnp.sum(acc_ref[1])
        for j in range(2, 16):
            stats_ref[0, j] = 0.0


_ROWS = 512


def _nll_stage(pred, gt):
    B, C, H, W = pred.shape
    rb = H // _ROWS
    nll, stats = pl.pallas_call(
        _nll_body,
        grid=(B * rb,),
        in_specs=[
            pl.BlockSpec((1, C, _ROWS, W),
                         lambda i: (i // rb, 0, i % rb, 0)),
            pl.BlockSpec((1, _ROWS, W), lambda i: (i // rb, i % rb, 0)),
        ],
        out_specs=[
            pl.BlockSpec((1, _ROWS, W), lambda i: (i // rb, i % rb, 0)),
            pl.BlockSpec(memory_space=pltpu.SMEM),
        ],
        out_shape=[
            jax.ShapeDtypeStruct((B, H, W), jnp.float32),
            jax.ShapeDtypeStruct((1, 16), jnp.float32),
        ],
        scratch_shapes=[pltpu.VMEM((2, _ROWS, W), jnp.float32)],
    )(pred, gt)
    return nll, stats


def _make_sc_select(shape3, topk_num):
    B, H, W = shape3
    npix = B * H * W
    n_tiles = 16
    nbuf = 4
    rows = 8
    chunk = rows * W
    nchunks = npix // chunk // n_tiles
    rows_per_b = H // rows
    kf = float(topk_num)

    mesh = plsc.VectorSubcoreMesh(core_axis_name="c", subcore_axis_name="s")

    def body(nll_hbm, stats_hbm, out_hbm,
             buf, ccnt, csum, tmp, acc, statv, outv,
             st_cnt, st_sum, sh_cnt, sh_sum, sem0, sem1, sem2, sem3):
        sid = lax.axis_index("s")
        cid = lax.axis_index("c")

        z16 = jnp.zeros((16,), jnp.float32)
        ones16 = jnp.ones((16,), jnp.float32)
        lane = lax.iota(jnp.int32, 16)

        def init_i(i, c):
            ccnt[pl.ds(i * 16, 16)] = z16
            csum[pl.ds(i * 16, 16)] = z16
            return c

        lax.fori_loop(0, 256, init_i, 0)

        def merge_hists(pairs, readback_all=True):
            for local, stage, merged in pairs:
                pltpu.sync_copy(local, stage.at[sid])
            plsc.subcore_barrier()
            part0 = sid * 256
            for local, stage, merged in pairs:
                cps = [pltpu.async_copy(
                           stage.at[tt, pl.ds(part0, 256)], tmp.at[tt],
                           sem0)
                       for tt in range(16)]
                for rr in range(16):
                    acc[pl.ds(rr * 16, 16)] = z16
                for tt in range(16):
                    cps[tt].wait()
                    for rr in range(16):
                        acc[pl.ds(rr * 16, 16)] = (
                            acc[pl.ds(rr * 16, 16)]
                            + tmp[tt, pl.ds(rr * 16, 16)])
                pltpu.sync_copy(acc, merged.at[sid])
            plsc.subcore_barrier()
            for local, stage, merged in pairs:
                if readback_all:
                    cps = [pltpu.async_copy(
                               merged.at[p], local.at[pl.ds(p * 256, 256)],
                               sem0)
                           for p in range(16)]
                    for p in range(16):
                        cps[p].wait()
                else:
                    @pl.when(sid == 0)
                    def _(local=local, merged=merged):
                        cps = [pltpu.async_copy(
                                   merged.at[p],
                                   local.at[pl.ds(p * 256, 256)], sem0)
                               for p in range(16)]
                        for p in range(16):
                            cps[p].wait()

        sems = (sem0, sem1, sem2, sem3)
        nouter = nchunks // nbuf

        def start_fetch(ci, bi):
            b = ci // rows_per_b
            r0 = (ci % rows_per_b) * rows
            return pltpu.async_copy(
                nll_hbm.at[b, pl.ds(r0, rows), :], buf.at[bi], sems[bi])

        def stream(process_vreg):
            for bi in range(nbuf):
                start_fetch(sid * nchunks + bi, bi)

            def outer(g, c):
                for bi in range(nbuf):
                    pltpu.make_async_copy(
                        nll_hbm.at[0, pl.ds(0, rows), :], buf.at[bi],
                        sems[bi]).wait()

                    @plsc.parallel_loop(0, chunk // 16, unroll=8)
                    def vloop(j, bi=bi):
                        process_vreg(
                            buf[bi, j >> 5, pl.ds((j & 31) * 16, 16)])

                    @pl.when(g < nouter - 1)
                    def _(bi=bi):
                        start_fetch(sid * nchunks + g * nbuf + bi + nbuf,
                                    bi)
                return c

            lax.fori_loop(0, nouter, outer, 0)

        def coarse_vreg(v):
            slot = ((plsc.bitcast(v, jnp.int32) >> 23) << 4) + lane
            plsc.addupdate_scatter(ccnt, [slot], ones16)
            plsc.addupdate_scatter(csum, [slot], v)

        with jax.named_scope("sc_coarse_stream"):
            stream(coarse_vreg)

        with jax.named_scope("sc_coarse_merge"):
            merge_hists([(ccnt, st_cnt, sh_cnt), (csum, st_sum, sh_sum)])

        def cscan(t, carry):
            acc_c, acc_s, b0, cg, sg = carry
            b = 255 - t
            tc = jnp.sum(ccnt[pl.ds(b * 16, 16)])
            ts = jnp.sum(csum[pl.ds(b * 16, 16)])
            found = b0 >= 0
            crossed = jnp.logical_and(jnp.logical_not(found),
                                      acc_c + tc >= kf)
            b0 = jnp.where(crossed, b, b0)
            cg = jnp.where(crossed, acc_c, cg)
            sg = jnp.where(crossed, acc_s, sg)
            keep = jnp.logical_or(found, crossed)
            acc_c = jnp.where(keep, acc_c, acc_c + tc)
            acc_s = jnp.where(keep, acc_s, acc_s + ts)
            return acc_c, acc_s, b0, cg, sg

        _, _, b0, cg, sg = lax.fori_loop(
            0, 256, cscan,
            (jnp.float32(0.0), jnp.float32(0.0), jnp.int32(-1),
             jnp.float32(0.0), jnp.float32(0.0)))

        fcnt, fsum = ccnt, csum

        def init_f(i, c):
            fcnt[pl.ds(i * 16, 16)] = z16
            fsum[pl.ds(i * 16, 16)] = z16
            return c

        lax.fori_loop(0, 256, init_f, 0)

        def fine_vreg(v):
            bits = plsc.bitcast(v, jnp.int32)
            m = (bits >> 23) == b0
            slot = (((bits >> 16) & 0x7F) << 4) + lane
            plsc.addupdate_scatter(fcnt, [slot], ones16, mask=m)
            plsc.addupdate_scatter(fsum, [slot], v, mask=m)

        with jax.named_scope("sc_fine_stream"):
            stream(fine_vreg)

        with jax.named_scope("sc_fine_merge"):
            merge_hists([(fcnt, st_cnt, sh_cnt), (fsum, st_sum, sh_sum)],
                        readback_all=False)

        @pl.when(jnp.logical_and(sid == 0, cid == 0))
        def _():

            def fscan(t, carry):
                acc_c, acc_s, f0, cg2, sg2, tot0, sum0 = carry
                f = 127 - t
                tc = jnp.sum(fcnt[pl.ds(f * 16, 16)])
                ts = jnp.sum(fsum[pl.ds(f * 16, 16)])
                found = f0 >= 0
                crossed = jnp.logical_and(jnp.logical_not(found),
                                          acc_c + tc >= kf)
                f0 = jnp.where(crossed, f, f0)
                cg2 = jnp.where(crossed, acc_c, cg2)
                sg2 = jnp.where(crossed, acc_s, sg2)
                tot0 = jnp.where(crossed, tc, tot0)
                sum0 = jnp.where(crossed, ts, sum0)
                keep = jnp.logical_or(found, crossed)
                acc_c = jnp.where(keep, acc_c, acc_c + tc)
                acc_s = jnp.where(keep, acc_s, acc_s + ts)
                return acc_c, acc_s, f0, cg2, sg2, tot0, sum0

            _, _, f0, cg2, sg2, tot0, sum0 = lax.fori_loop(
                0, 128, fscan,
                (cg, sg, jnp.int32(-1), cg, sg,
                 jnp.float32(0.0), jnp.float32(0.0)))

            pltpu.sync_copy(stats_hbm.at[0], statv)
            statv_v = statv[...]
            b16 = lambda x: jnp.broadcast_to(x, (16,))
            r16 = b16(kf - cg2)
            binmean16 = b16(sum0) / jnp.maximum(b16(tot0), 1.0)
            topk16 = b16(sg2) + r16 * binmean16
            total16 = b16(statv_v[0])
            valid16 = b16(statv_v[1])
            loss16 = total16 / (valid16 + 1e-12) \
                + topk16 * (TOP_WEIGHT / kf)
            outv[...] = loss16
            pltpu.sync_copy(outv, out_hbm)

    return functools.partial(
        pl.kernel, body,
        out_type=jax.ShapeDtypeStruct((16,), jnp.float32),
        mesh=mesh,
        compiler_params=pltpu.CompilerParams(needs_layout_passes=False),
        scratch_types=[
            pltpu.VMEM((nbuf, rows, W), jnp.float32),
            pltpu.VMEM((4096,), jnp.float32),
            pltpu.VMEM((4096,), jnp.float32),
            pltpu.VMEM((16, 256), jnp.float32),
            pltpu.VMEM((256,), jnp.float32),
            pltpu.VMEM((16,), jnp.float32),
            pltpu.VMEM((16,), jnp.float32),
            pltpu.HBM((16, 4096), jnp.float32),
            pltpu.HBM((16, 4096), jnp.float32),
            pltpu.HBM((16, 256), jnp.float32),
            pltpu.HBM((16, 256), jnp.float32),
            pltpu.SemaphoreType.DMA,
            pltpu.SemaphoreType.DMA,
            pltpu.SemaphoreType.DMA,
            pltpu.SemaphoreType.DMA,
        ],
    )()


def kernel(pred, gt):
    topk_num = int(gt.size * TOP_RATIO)
    nll, stats = _nll_stage(pred, gt)
    loss16 = _make_sc_select(nll.shape, topk_num)(nll, stats)
    return loss16[0].reshape(())


# speedup vs baseline: 1.0135x; 1.0135x over previous
import functools

import jax
import jax.numpy as jnp
from jax import lax
from jax.experimental import pallas as pl
from jax.experimental.pallas import tpu as pltpu
from jax.experimental.pallas import tpu_sc as plsc

TOP_RATIO = 0.3
TOP_WEIGHT = 1.0
IGNORE_INDEX = -1

_PIX_BLK = 8192


def _nll_body(pred_ref, gt_ref, nll_ref, stats_ref, acc_ref):
    i = pl.program_id(0)
    n = pl.num_programs(0)
    x = pred_ref[0]
    g = gt_ref[0]
    s = jnp.sum(jnp.exp(x), axis=0)
    lse = jnp.log(s)
    cio = lax.broadcasted_iota(jnp.int32, x.shape, 0)
    xg = jnp.sum(jnp.where(cio == g[None], x, 0.0), axis=0)
    valid = g != IGNORE_INDEX
    nll = jnp.where(valid, jnp.maximum(lse - xg, 0.0), 0.0)
    nll_ref[0] = nll

    @pl.when(i == 0)
    def _():
        acc_ref[0] = jnp.zeros_like(acc_ref[0])
        acc_ref[1] = jnp.zeros_like(acc_ref[1])

    acc_ref[0] += nll
    acc_ref[1] += valid.astype(jnp.float32)

    @pl.when(i == n - 1)
    def _():
        stats_ref[0, 0] = jnp.sum(acc_ref[0])
        stats_ref[0, 1] = jnp.sum(acc_ref[1])
        for j in range(2, 16):
            stats_ref[0, j] = 0.0


_ROWS = 256


def _nll_stage(pred, gt):
    B, C, H, W = pred.shape
    rb = H // _ROWS
    nll, stats = pl.pallas_call(
        _nll_body,
        grid=(B * rb,),
        in_specs=[
            pl.BlockSpec((1, C, _ROWS, W),
                         lambda i: (i // rb, 0, i % rb, 0)),
            pl.BlockSpec((1, _ROWS, W), lambda i: (i // rb, i % rb, 0)),
        ],
        out_specs=[
            pl.BlockSpec((1, _ROWS, W), lambda i: (i // rb, i % rb, 0)),
            pl.BlockSpec(memory_space=pltpu.SMEM),
        ],
        out_shape=[
            jax.ShapeDtypeStruct((B, H, W), jnp.float32),
            jax.ShapeDtypeStruct((1, 16), jnp.float32),
        ],
        scratch_shapes=[pltpu.VMEM((2, _ROWS, W), jnp.float32)],
    )(pred, gt)
    return nll, stats


def _make_sc_select(shape3, topk_num):
    B, H, W = shape3
    npix = B * H * W
    n_tiles = 16
    nbuf = 4
    rows = 8
    chunk = rows * W
    nchunks = npix // chunk // n_tiles
    rows_per_b = H // rows
    kf = float(topk_num)

    mesh = plsc.VectorSubcoreMesh(core_axis_name="c", subcore_axis_name="s")

    def body(nll_hbm, stats_hbm, out_hbm,
             buf, ccnt, csum, tmp, acc, statv, outv,
             st_cnt, st_sum, sh_cnt, sh_sum, sem0, sem1, sem2, sem3):
        sid = lax.axis_index("s")
        cid = lax.axis_index("c")

        z16 = jnp.zeros((16,), jnp.float32)
        ones16 = jnp.ones((16,), jnp.float32)
        lane = lax.iota(jnp.int32, 16)

        def init_i(i, c):
            ccnt[pl.ds(i * 16, 16)] = z16
            csum[pl.ds(i * 16, 16)] = z16
            return c

        lax.fori_loop(0, 256, init_i, 0)

        def merge_hists(pairs, readback_all=True):
            for local, stage, merged in pairs:
                pltpu.sync_copy(local, stage.at[sid])
            plsc.subcore_barrier()
            part0 = sid * 256
            for local, stage, merged in pairs:
                cps = [pltpu.async_copy(
                           stage.at[tt, pl.ds(part0, 256)], tmp.at[tt],
                           sem0)
                       for tt in range(16)]
                for rr in range(16):
                    acc[pl.ds(rr * 16, 16)] = z16
                for tt in range(16):
                    cps[tt].wait()
                    for rr in range(16):
                        acc[pl.ds(rr * 16, 16)] = (
                            acc[pl.ds(rr * 16, 16)]
                            + tmp[tt, pl.ds(rr * 16, 16)])
                pltpu.sync_copy(acc, merged.at[sid])
            plsc.subcore_barrier()
            for local, stage, merged in pairs:
                if readback_all:
                    cps = [pltpu.async_copy(
                               merged.at[p], local.at[pl.ds(p * 256, 256)],
                               sem0)
                           for p in range(16)]
                    for p in range(16):
                        cps[p].wait()
                else:
                    @pl.when(sid == 0)
                    def _(local=local, merged=merged):
                        cps = [pltpu.async_copy(
                                   merged.at[p],
                                   local.at[pl.ds(p * 256, 256)], sem0)
                               for p in range(16)]
                        for p in range(16):
                            cps[p].wait()

        sems = (sem0, sem1, sem2, sem3)
        nouter = nchunks // nbuf

        def start_fetch(ci, bi):
            b = ci // rows_per_b
            r0 = (ci % rows_per_b) * rows
            return pltpu.async_copy(
                nll_hbm.at[b, pl.ds(r0, rows), :], buf.at[bi], sems[bi])

        def stream(process_vreg):
            for bi in range(nbuf):
                start_fetch(sid * nchunks + bi, bi)

            def outer(g, c):
                for bi in range(nbuf):
                    pltpu.make_async_copy(
                        nll_hbm.at[0, pl.ds(0, rows), :], buf.at[bi],
                        sems[bi]).wait()

                    @plsc.parallel_loop(0, chunk // 16, unroll=8)
                    def vloop(j, bi=bi):
                        process_vreg(
                            buf[bi, j >> 5, pl.ds((j & 31) * 16, 16)])

                    @pl.when(g < nouter - 1)
                    def _(bi=bi):
                        start_fetch(sid * nchunks + g * nbuf + bi + nbuf,
                                    bi)
                return c

            lax.fori_loop(0, nouter, outer, 0)

        def coarse_vreg(v):
            slot = ((plsc.bitcast(v, jnp.int32) >> 23) << 4) + lane
            plsc.addupdate_scatter(ccnt, [slot], ones16)
            plsc.addupdate_scatter(csum, [slot], v)

        stream(coarse_vreg)

        merge_hists([(ccnt, st_cnt, sh_cnt), (csum, st_sum, sh_sum)])

        def cscan(t, carry):
            acc_c, acc_s, b0, cg, sg = carry
            b = 255 - t
            tc = jnp.sum(ccnt[pl.ds(b * 16, 16)])
            ts = jnp.sum(csum[pl.ds(b * 16, 16)])
            found = b0 >= 0
            crossed = jnp.logical_and(jnp.logical_not(found),
                                      acc_c + tc >= kf)
            b0 = jnp.where(crossed, b, b0)
            cg = jnp.where(crossed, acc_c, cg)
            sg = jnp.where(crossed, acc_s, sg)
            keep = jnp.logical_or(found, crossed)
            acc_c = jnp.where(keep, acc_c, acc_c + tc)
            acc_s = jnp.where(keep, acc_s, acc_s + ts)
            return acc_c, acc_s, b0, cg, sg

        _, _, b0, cg, sg = lax.fori_loop(
            0, 256, cscan,
            (jnp.float32(0.0), jnp.float32(0.0), jnp.int32(-1),
             jnp.float32(0.0), jnp.float32(0.0)))

        fcnt, fsum = ccnt, csum

        def init_f(i, c):
            fcnt[pl.ds(i * 16, 16)] = z16
            fsum[pl.ds(i * 16, 16)] = z16
            return c

        lax.fori_loop(0, 256, init_f, 0)

        def fine_vreg(v):
            bits = plsc.bitcast(v, jnp.int32)
            m = (bits >> 23) == b0
            slot = (((bits >> 16) & 0x7F) << 4) + lane
            plsc.addupdate_scatter(fcnt, [slot], ones16, mask=m)
            plsc.addupdate_scatter(fsum, [slot], v, mask=m)

        stream(fine_vreg)

        merge_hists([(fcnt, st_cnt, sh_cnt), (fsum, st_sum, sh_sum)],
                    readback_all=False)

        @pl.when(jnp.logical_and(sid == 0, cid == 0))
        def _():

            def fscan(t, carry):
                acc_c, acc_s, f0, cg2, sg2, tot0, sum0 = carry
                f = 127 - t
                tc = jnp.sum(fcnt[pl.ds(f * 16, 16)])
                ts = jnp.sum(fsum[pl.ds(f * 16, 16)])
                found = f0 >= 0
                crossed = jnp.logical_and(jnp.logical_not(found),
                                          acc_c + tc >= kf)
                f0 = jnp.where(crossed, f, f0)
                cg2 = jnp.where(crossed, acc_c, cg2)
                sg2 = jnp.where(crossed, acc_s, sg2)
                tot0 = jnp.where(crossed, tc, tot0)
                sum0 = jnp.where(crossed, ts, sum0)
                keep = jnp.logical_or(found, crossed)
                acc_c = jnp.where(keep, acc_c, acc_c + tc)
                acc_s = jnp.where(keep, acc_s, acc_s + ts)
                return acc_c, acc_s, f0, cg2, sg2, tot0, sum0

            _, _, f0, cg2, sg2, tot0, sum0 = lax.fori_loop(
                0, 128, fscan,
                (cg, sg, jnp.int32(-1), cg, sg,
                 jnp.float32(0.0), jnp.float32(0.0)))

            pltpu.sync_copy(stats_hbm.at[0], statv)
            statv_v = statv[...]
            b16 = lambda x: jnp.broadcast_to(x, (16,))
            r16 = b16(kf - cg2)
            binmean16 = b16(sum0) / jnp.maximum(b16(tot0), 1.0)
            topk16 = b16(sg2) + r16 * binmean16
            total16 = b16(statv_v[0])
            valid16 = b16(statv_v[1])
            loss16 = total16 / (valid16 + 1e-12) \
                + topk16 * (TOP_WEIGHT / kf)
            outv[...] = loss16
            pltpu.sync_copy(outv, out_hbm)

    return functools.partial(
        pl.kernel, body,
        out_type=jax.ShapeDtypeStruct((16,), jnp.float32),
        mesh=mesh,
        compiler_params=pltpu.CompilerParams(needs_layout_passes=False),
        scratch_types=[
            pltpu.VMEM((nbuf, rows, W), jnp.float32),
            pltpu.VMEM((4096,), jnp.float32),
            pltpu.VMEM((4096,), jnp.float32),
            pltpu.VMEM((16, 256), jnp.float32),
            pltpu.VMEM((256,), jnp.float32),
            pltpu.VMEM((16,), jnp.float32),
            pltpu.VMEM((16,), jnp.float32),
            pltpu.HBM((16, 4096), jnp.float32),
            pltpu.HBM((16, 4096), jnp.float32),
            pltpu.HBM((16, 256), jnp.float32),
            pltpu.HBM((16, 256), jnp.float32),
            pltpu.SemaphoreType.DMA,
            pltpu.SemaphoreType.DMA,
            pltpu.SemaphoreType.DMA,
            pltpu.SemaphoreType.DMA,
        ],
    )()


def kernel(pred, gt):
    topk_num = int(gt.size * TOP_RATIO)
    nll, stats = _nll_stage(pred, gt)
    loss16 = _make_sc_select(nll.shape, topk_num)(nll, stats)
    return loss16[0].reshape(())


# speedup vs baseline: 1.0149x; 1.0013x over previous
import functools

import jax
import jax.numpy as jnp
from jax import lax
from jax.experimental import pallas as pl
from jax.experimental.pallas import tpu as pltpu
from jax.experimental.pallas import tpu_sc as plsc

TOP_RATIO = 0.3
TOP_WEIGHT = 1.0
IGNORE_INDEX = -1

def _nll_body(pred_ref, gt_ref, nll_ref, stats_ref, acc_ref):
    i = pl.program_id(0)
    n = pl.num_programs(0)
    x = pred_ref[0]
    g = gt_ref[0]
    s = jnp.sum(jnp.exp(x), axis=0)
    lse = jnp.log(s)
    cio = lax.broadcasted_iota(jnp.int32, x.shape, 0)
    xg = jnp.sum(jnp.where(cio == g[None], x, 0.0), axis=0)
    valid = g != IGNORE_INDEX
    nll = jnp.where(valid, jnp.maximum(lse - xg, 0.0), 0.0)
    nll_ref[0] = nll

    @pl.when(i == 0)
    def _():
        acc_ref[0] = jnp.zeros_like(acc_ref[0])
        acc_ref[1] = jnp.zeros_like(acc_ref[1])

    acc_ref[0] += nll
    acc_ref[1] += valid.astype(jnp.float32)

    @pl.when(i == n - 1)
    def _():
        stats_ref[0, 0] = jnp.sum(acc_ref[0])
        stats_ref[0, 1] = jnp.sum(acc_ref[1])
        for j in range(2, 16):
            stats_ref[0, j] = 0.0


_ROWS = 256


def _nll_stage(pred, gt):
    B, C, H, W = pred.shape
    rb = H // _ROWS
    nll, stats = pl.pallas_call(
        _nll_body,
        grid=(B * rb,),
        in_specs=[
            pl.BlockSpec((1, C, _ROWS, W),
                         lambda i: (i // rb, 0, i % rb, 0)),
            pl.BlockSpec((1, _ROWS, W), lambda i: (i // rb, i % rb, 0)),
        ],
        out_specs=[
            pl.BlockSpec((1, _ROWS, W), lambda i: (i // rb, i % rb, 0)),
            pl.BlockSpec(memory_space=pltpu.SMEM),
        ],
        out_shape=[
            jax.ShapeDtypeStruct((B, H, W), jnp.float32),
            jax.ShapeDtypeStruct((1, 16), jnp.float32),
        ],
        scratch_shapes=[pltpu.VMEM((2, _ROWS, W), jnp.float32)],
    )(pred, gt)
    return nll, stats


def _make_sc_select(shape3, topk_num):
    B, H, W = shape3
    npix = B * H * W
    n_tiles = 16
    nbuf = 4
    rows = 8
    chunk = rows * W
    nchunks = npix // chunk // n_tiles
    rows_per_b = H // rows
    kf = float(topk_num)

    mesh = plsc.VectorSubcoreMesh(core_axis_name="c", subcore_axis_name="s")

    def body(nll_hbm, stats_hbm, out_hbm,
             buf, ccnt, csum, tmp, acc, statv, outv,
             st_cnt, st_sum, sh_cnt, sh_sum, sem0, sem1, sem2, sem3):
        sid = lax.axis_index("s")
        cid = lax.axis_index("c")

        z16 = jnp.zeros((16,), jnp.float32)
        ones16 = jnp.ones((16,), jnp.float32)
        lane = lax.iota(jnp.int32, 16)

        def init_i(i, c):
            ccnt[pl.ds(i * 16, 16)] = z16
            csum[pl.ds(i * 16, 16)] = z16
            return c

        lax.fori_loop(0, 256, init_i, 0)

        def merge_hists(pairs, readback_all=True):
            for local, stage, merged in pairs:
                pltpu.sync_copy(local, stage.at[sid])
            plsc.subcore_barrier()
            part0 = sid * 256
            for local, stage, merged in pairs:
                cps = [pltpu.async_copy(
                           stage.at[tt, pl.ds(part0, 256)], tmp.at[tt],
                           sem0)
                       for tt in range(16)]
                for rr in range(16):
                    acc[pl.ds(rr * 16, 16)] = z16
                for tt in range(16):
                    cps[tt].wait()
                    for rr in range(16):
                        acc[pl.ds(rr * 16, 16)] = (
                            acc[pl.ds(rr * 16, 16)]
                            + tmp[tt, pl.ds(rr * 16, 16)])
                pltpu.sync_copy(acc, merged.at[sid])
            plsc.subcore_barrier()
            for local, stage, merged in pairs:
                if readback_all:
                    cps = [pltpu.async_copy(
                               merged.at[p], local.at[pl.ds(p * 256, 256)],
                               sem0)
                           for p in range(16)]
                    for p in range(16):
                        cps[p].wait()
                else:
                    @pl.when(sid == 0)
                    def _(local=local, merged=merged):
                        cps = [pltpu.async_copy(
                                   merged.at[p],
                                   local.at[pl.ds(p * 256, 256)], sem0)
                               for p in range(16)]
                        for p in range(16):
                            cps[p].wait()

        sems = (sem0, sem1, sem2, sem3)
        nouter = nchunks // nbuf

        def start_fetch(ci, bi):
            b = ci // rows_per_b
            r0 = (ci % rows_per_b) * rows
            return pltpu.async_copy(
                nll_hbm.at[b, pl.ds(r0, rows), :], buf.at[bi], sems[bi])

        def stream(process_vreg):
            for bi in range(nbuf):
                start_fetch(sid * nchunks + bi, bi)

            def outer(g, c):
                for bi in range(nbuf):
                    pltpu.make_async_copy(
                        nll_hbm.at[0, pl.ds(0, rows), :], buf.at[bi],
                        sems[bi]).wait()

                    @plsc.parallel_loop(0, chunk // 16, unroll=8)
                    def vloop(j, bi=bi):
                        process_vreg(
                            buf[bi, j >> 5, pl.ds((j & 31) * 16, 16)])

                    @pl.when(g < nouter - 1)
                    def _(bi=bi):
                        start_fetch(sid * nchunks + g * nbuf + bi + nbuf,
                                    bi)
                return c

            lax.fori_loop(0, nouter, outer, 0)

        def coarse_vreg(v):
            slot = ((plsc.bitcast(v, jnp.int32) >> 23) << 4) + lane
            plsc.addupdate_scatter(ccnt, [slot], ones16)
            plsc.addupdate_scatter(csum, [slot], v)

        stream(coarse_vreg)

        merge_hists([(ccnt, st_cnt, sh_cnt), (csum, st_sum, sh_sum)])

        def cscan(t, carry):
            acc_c, acc_s, b0, cg, sg = carry
            b = 255 - t
            tc = jnp.sum(ccnt[pl.ds(b * 16, 16)])
            ts = jnp.sum(csum[pl.ds(b * 16, 16)])
            found = b0 >= 0
            crossed = jnp.logical_and(jnp.logical_not(found),
                                      acc_c + tc >= kf)
            b0 = jnp.where(crossed, b, b0)
            cg = jnp.where(crossed, acc_c, cg)
            sg = jnp.where(crossed, acc_s, sg)
            keep = jnp.logical_or(found, crossed)
            acc_c = jnp.where(keep, acc_c, acc_c + tc)
            acc_s = jnp.where(keep, acc_s, acc_s + ts)
            return acc_c, acc_s, b0, cg, sg

        _, _, b0, cg, sg = lax.fori_loop(
            0, 256, cscan,
            (jnp.float32(0.0), jnp.float32(0.0), jnp.int32(-1),
             jnp.float32(0.0), jnp.float32(0.0)))

        fcnt, fsum = ccnt, csum

        def init_f(i, c):
            fcnt[pl.ds(i * 16, 16)] = z16
            fsum[pl.ds(i * 16, 16)] = z16
            return c

        lax.fori_loop(0, 256, init_f, 0)

        def fine_vreg(v):
            bits = plsc.bitcast(v, jnp.int32)
            m = (bits >> 23) == b0
            slot = (((bits >> 16) & 0x7F) << 4) + lane
            plsc.addupdate_scatter(fcnt, [slot], ones16, mask=m)
            plsc.addupdate_scatter(fsum, [slot], v, mask=m)

        stream(fine_vreg)

        merge_hists([(fcnt, st_cnt, sh_cnt), (fsum, st_sum, sh_sum)],
                    readback_all=False)

        @pl.when(jnp.logical_and(sid == 0, cid == 0))
        def _():

            def fscan(t, carry):
                acc_c, acc_s, f0, cg2, sg2, tot0, sum0 = carry
                f = 127 - t
                tc = jnp.sum(fcnt[pl.ds(f * 16, 16)])
                ts = jnp.sum(fsum[pl.ds(f * 16, 16)])
                found = f0 >= 0
                crossed = jnp.logical_and(jnp.logical_not(found),
                                          acc_c + tc >= kf)
                f0 = jnp.where(crossed, f, f0)
                cg2 = jnp.where(crossed, acc_c, cg2)
                sg2 = jnp.where(crossed, acc_s, sg2)
                tot0 = jnp.where(crossed, tc, tot0)
                sum0 = jnp.where(crossed, ts, sum0)
                keep = jnp.logical_or(found, crossed)
                acc_c = jnp.where(keep, acc_c, acc_c + tc)
                acc_s = jnp.where(keep, acc_s, acc_s + ts)
                return acc_c, acc_s, f0, cg2, sg2, tot0, sum0

            _, _, f0, cg2, sg2, tot0, sum0 = lax.fori_loop(
                0, 128, fscan,
                (cg, sg, jnp.int32(-1), cg, sg,
                 jnp.float32(0.0), jnp.float32(0.0)))

            pltpu.sync_copy(stats_hbm.at[0], statv)
            statv_v = statv[...]
            b16 = lambda x: jnp.broadcast_to(x, (16,))
            r16 = b16(kf - cg2)
            binmean16 = b16(sum0) / jnp.maximum(b16(tot0), 1.0)
            topk16 = b16(sg2) + r16 * binmean16
            total16 = b16(statv_v[0])
            valid16 = b16(statv_v[1])
            loss16 = total16 / (valid16 + 1e-12) \
                + topk16 * (TOP_WEIGHT / kf)
            outv[...] = loss16
            pltpu.sync_copy(outv, out_hbm)

    return functools.partial(
        pl.kernel, body,
        out_type=jax.ShapeDtypeStruct((16,), jnp.float32),
        mesh=mesh,
        compiler_params=pltpu.CompilerParams(needs_layout_passes=False),
        scratch_types=[
            pltpu.VMEM((nbuf, rows, W), jnp.float32),
            pltpu.VMEM((4096,), jnp.float32),
            pltpu.VMEM((4096,), jnp.float32),
            pltpu.VMEM((16, 256), jnp.float32),
            pltpu.VMEM((256,), jnp.float32),
            pltpu.VMEM((16,), jnp.float32),
            pltpu.VMEM((16,), jnp.float32),
            pltpu.HBM((16, 4096), jnp.float32),
            pltpu.HBM((16, 4096), jnp.float32),
            pltpu.HBM((16, 256), jnp.float32),
            pltpu.HBM((16, 256), jnp.float32),
            pltpu.SemaphoreType.DMA,
            pltpu.SemaphoreType.DMA,
            pltpu.SemaphoreType.DMA,
            pltpu.SemaphoreType.DMA,
        ],
    )()


def kernel(pred, gt):
    topk_num = int(gt.size * TOP_RATIO)
    nll, stats = _nll_stage(pred, gt)
    loss16 = _make_sc_select(nll.shape, topk_num)(nll, stats)
    return loss16[0].reshape(())
